# Initial kernel scaffold; baseline (speedup 1.0000x reference)
import jax
import jax.numpy as jnp
from jax.experimental import pallas as pl


def kernel(inputs):
    raise NotImplementedError("write your pallas kernel here")


# speedup vs baseline: 2.2798x; 2.2798x over previous
import functools

import numpy as np
import jax
import jax.numpy as jnp
from jax.experimental import pallas as pl
from jax.experimental.pallas import tpu as pltpu
from jax.experimental.pallas import tpu_sc as plsc

B, H, W, C = 2, 384, 384, 96
P = 110592
K = 27648
RAND = 9216
BP = B * P
R4 = 4 * BP
NW = 32
PERW = R4 // NW
CHUNK = 128
NCH = PERW // CHUNK
CPAD = 128
TP = 2048
NB = BP // TP
NROW = P // 128
NPAD = 1024
N_SORT = NPAD * 128
KROWS = K // 128


def _raw_constants():
    def build():
        key = jax.random.key(42)
        k1, k2 = jax.random.split(key)
        pc = jax.random.uniform(k1, (B, P, 2), dtype=jnp.float32)
        rand_pts = jax.random.uniform(k2, (B, RAND, 2), dtype=jnp.float32)
        grid = pc * 2.0 - 1.0
        gx, gy = grid[..., 0], grid[..., 1]
        x = (gx + 1.0) * W / 2.0 - 0.5
        y = (gy + 1.0) * H / 2.0 - 0.5
        x0 = jnp.floor(x)
        y0 = jnp.floor(y)
        x1 = x0 + 1.0
        y1 = y0 + 1.0
        wx1 = x - x0
        wx0 = 1.0 - wx1
        wy1 = y - y0
        wy0 = 1.0 - wy1
        offs, ws = [], []
        for (xi, yi, wx, wy) in ((x0, y0, wx0, wy0), (x1, y0, wx1, wy0),
                                 (x0, y1, wx0, wy1), (x1, y1, wx1, wy1)):
            valid = (xi >= 0) & (xi <= W - 1) & (yi >= 0) & (yi <= H - 1)
            xi_c = jnp.clip(xi, 0, W - 1).astype(jnp.int32)
            yi_c = jnp.clip(yi, 0, H - 1).astype(jnp.int32)
            offs.append(yi_c * W + xi_c)
            ws.append(jnp.where(valid, wx * wy, 0.0))
        boff = (jnp.arange(B, dtype=jnp.int32) * (H * W))[:, None]
        offs = jnp.stack(offs, 0) + boff[None]
        ws = jnp.stack(ws, 0)
        return pc, rand_pts, offs, ws

    return build


def _derive(pc, rand_pts, offs, ws):
    np_ = jnp if isinstance(pc, jax.core.Tracer) else np
    idx_flat = offs.astype(np_.int32).reshape(R4)
    wts = np_.transpose(ws.astype(np_.float32).reshape(4, BP))
    pad = np_.zeros((B, NPAD - NROW, 128), np_.float32)
    px = np_.concatenate(
        [pc[..., 0].astype(np_.float32).reshape(B, NROW, 128), pad], axis=1)
    py = np_.concatenate(
        [pc[..., 1].astype(np_.float32).reshape(B, NROW, 128), pad], axis=1)
    return idx_flat, wts, px, py, rand_pts.astype(np_.float32)


def _host_constants():
    try:
        cpu = jax.devices("cpu")[0]
        with jax.default_device(cpu):
            vals = jax.device_get(jax.jit(_raw_constants())())
    except Exception:
        return None
    return _derive(*(np.asarray(v) for v in vals))


_CONSTS = _host_constants()


def _constants():
    return _derive(*_raw_constants()())


def _gather_body(feat_hbm, idx_hbm, out_hbm, idx_v, buf0, buf1, sem0, sem1):
    cid = jax.lax.axis_index("c")
    sid = jax.lax.axis_index("s")
    wid = sid * 2 + cid
    base = wid * PERW
    pltpu.sync_copy(idx_hbm.at[pl.ds(base, PERW)], idx_v)

    def start(c, buf, sem):
        cpy = pltpu.make_async_copy(
            feat_hbm.at[idx_v.at[pl.ds(c * CHUNK, CHUNK)]], buf, sem)
        cpy.start()

    def wait(buf, sem):
        pltpu.make_async_copy(feat_hbm.at[idx_v.at[pl.ds(0, CHUNK)]],
                              buf, sem).wait()

    start(0, buf0, sem0)
    start(1, buf1, sem1)

    def body(i, _):
        c0 = 2 * i
        wait(buf0, sem0)
        pltpu.sync_copy(buf0, out_hbm.at[pl.ds(base + c0 * CHUNK, CHUNK)])

        @pl.when(i < NCH // 2 - 1)
        def _():
            start(c0 + 2, buf0, sem0)

        wait(buf1, sem1)
        pltpu.sync_copy(buf1, out_hbm.at[pl.ds(base + (c0 + 1) * CHUNK, CHUNK)])

        @pl.when(i < NCH // 2 - 1)
        def _():
            start(c0 + 3, buf1, sem1)

        return 0

    jax.lax.fori_loop(0, NCH // 2, body, 0)


@functools.lru_cache(maxsize=1)
def _make_sc_gather():
    @functools.partial(
        pl.kernel,
        mesh=plsc.VectorSubcoreMesh(core_axis_name="c", subcore_axis_name="s"),
        out_type=jax.ShapeDtypeStruct((R4, CPAD), jnp.float32),
        scratch_types=[
            pltpu.VMEM((PERW,), jnp.int32),
            pltpu.VMEM((CHUNK, CPAD), jnp.float32),
            pltpu.VMEM((CHUNK, CPAD), jnp.float32),
            pltpu.SemaphoreType.DMA,
            pltpu.SemaphoreType.DMA,
        ],
    )
    def _sc_gather(feat_hbm, idx_hbm, out_hbm, idx_v, buf0, buf1, sem0, sem1):
        _gather_body(feat_hbm, idx_hbm, out_hbm, idx_v, buf0, buf1, sem0, sem1)

    return _sc_gather


def _rotl8(t, d):
    return jnp.concatenate([t[:, d:], t[:, :d]], axis=-1)


def _uncert_body(g_ref, w_ref, unc_ref):
    g = g_ref[..., :C]
    wv = w_ref[...]
    l = g[0] * wv[:, 0:1]
    l = l + g[1] * wv[:, 1:2]
    l = l + g[2] * wv[:, 2:3]
    l = l + g[3] * wv[:, 3:4]
    m = jnp.max(l, axis=-1, keepdims=True)
    u = jnp.exp(l - m)
    A = None
    for k in range(12):
        blk = u[:, k * 8:(k + 1) * 8]
        A = blk if A is None else A + blk
    for d in (4, 2, 1):
        A = A + _rotl8(A, d)
    p = u * pl.reciprocal(A[:, 0:1], approx=True)
    p1 = jnp.max(p, axis=-1)
    eq = p == p1[:, None]
    pos = jax.lax.broadcasted_iota(jnp.int32, p.shape, 1)
    first_idx = jnp.min(jnp.where(eq, pos, p.shape[-1]), axis=-1)
    first = pos == first_idx[:, None]
    p2 = jnp.max(jnp.where(first, -jnp.inf, p), axis=-1)
    unc_ref[...] = (p2 - p1).reshape(8, TP // 8)


def _tc_uncert(gath4, wts):
    return pl.pallas_call(
        _uncert_body,
        grid=(NB,),
        in_specs=[
            pl.BlockSpec((4, TP, CPAD), lambda i: (0, i, 0)),
            pl.BlockSpec((TP, 4), lambda i: (i, 0)),
        ],
        out_specs=pl.BlockSpec((8, TP // 8), lambda i: (i, 0)),
        out_shape=jax.ShapeDtypeStruct((NB * 8, TP // 8), jnp.float32),
    )(gath4, wts)


def _ce(h0, h1, desc, active=None):
    v0, i0 = h0[0], h0[1]
    v1, i1 = h1[0], h1[1]
    gt = (v0 > v1) | ((v0 == v1) & (i0 < i1))
    exch = jnp.logical_xor(gt, desc)
    if active is not None:
        exch = exch & active
    o0, o1 = [], []
    for a0, a1 in zip(h0, h1):
        o0.append(jnp.where(exch, a1, a0))
        o1.append(jnp.where(exch, a0, a1))
    return o0, o1


def _ce_rows(arrs, jr, k, active=None):
    G = NPAD // (2 * jr)
    rs = [a.reshape(G, 2, jr, 128) for a in arrs]
    h0 = [r[:, 0] for r in rs]
    h1 = [r[:, 1] for r in rs]
    g_iota = jax.lax.broadcasted_iota(jnp.int32, (G, 1, 1), 0)
    desc = ((g_iota * (2 * jr)) & jax.lax.shift_right_logical(k, 7)) == 0
    o0, o1 = _ce(h0, h1, desc, active)
    return [jnp.stack([a0, a1], axis=1).reshape(NPAD, 128)
            for a0, a1 in zip(o0, o1)]


def _ce_lanes(arrs_t, j, k, in_merge):
    G = 128 // (2 * j)
    rs = [a.reshape(G, 2, j, NPAD) for a in arrs_t]
    h0 = [r[:, 0] for r in rs]
    h1 = [r[:, 1] for r in rs]
    if in_merge:
        r_iota = jax.lax.broadcasted_iota(jnp.int32, (1, 1, NPAD), 2)
        desc = (r_iota & jax.lax.shift_right_logical(k, 7)) == 0
    else:
        g_iota = jax.lax.broadcasted_iota(jnp.int32, (G, 1, 1), 0)
        desc = ((g_iota * (2 * j)) & k) == 0
    o0, o1 = _ce(h0, h1, desc)
    return [jnp.stack([a0, a1], axis=1).reshape(128, NPAD)
            for a0, a1 in zip(o0, o1)]


def _sort_body(v_ref, x_ref, y_ref, ox_ref, oy_ref):
    v = v_ref[...]
    x = x_ref[...]
    y = y_ref[...]
    r_iota = jax.lax.broadcasted_iota(jnp.int32, (NPAD, 128), 0)
    c_iota = jax.lax.broadcasted_iota(jnp.int32, (NPAD, 128), 1)
    idx = r_iota * 128 + c_iota
    arrs_t = [jnp.transpose(a) for a in (v, idx, x, y)]

    k = 2
    while k <= 128:
        j = k // 2
        while j >= 1:
            arrs_t = _ce_lanes(arrs_t, j, k, in_merge=(k >= 128))
            j //= 2
        k *= 2

    def phase(p, carry):
        arrs_t = list(carry)
        kk = jax.lax.shift_left(1, p)
        arrs = [jnp.transpose(a) for a in arrs_t]
        for jr_log in range(9, -1, -1):
            jr = 1 << jr_log
            active = (2 * jr * 128) <= kk
            arrs = _ce_rows(arrs, jr, kk, active)
        arrs_t = [jnp.transpose(a) for a in arrs]
        for j in (64, 32, 16, 8, 4, 2, 1):
            arrs_t = _ce_lanes(arrs_t, j, kk, in_merge=True)
        return tuple(arrs_t)

    arrs_t = list(jax.lax.fori_loop(8, 18, phase, tuple(arrs_t)))

    xs = jnp.transpose(arrs_t[2])
    ys = jnp.transpose(arrs_t[3])
    ox_ref[...] = xs[:KROWS]
    oy_ref[...] = ys[:KROWS]


def _tc_sort(v, x, y):
    return pl.pallas_call(
        _sort_body,
        grid=(B,),
        in_specs=[pl.BlockSpec((None, NPAD, 128), lambda b: (b, 0, 0))] * 3,
        out_specs=[pl.BlockSpec((None, KROWS, 128), lambda b: (b, 0, 0))] * 2,
        out_shape=[jax.ShapeDtypeStruct((B, KROWS, 128), jnp.float32)] * 2,
    )(v, x, y)


def kernel(inputs):
    idx, wts, px, py, rand_pts = _constants()
    feat2d = inputs.reshape(B * H * W, C)
    featp = jnp.concatenate(
        [feat2d, jnp.zeros((B * H * W, CPAD - C), jnp.float32)], axis=-1)
    gath = _make_sc_gather()(featp, jnp.asarray(idx))
    gath4 = gath.reshape(4, BP, CPAD)
    unc = _tc_uncert(gath4, jnp.asarray(wts))
    v = unc.reshape(B, NROW, 128)
    v = jnp.concatenate(
        [v, jnp.full((B, NPAD - NROW, 128), -jnp.inf, jnp.float32)], axis=1)
    tx, ty = _tc_sort(v, jnp.asarray(px), jnp.asarray(py))
    top = jnp.stack([tx.reshape(B, K), ty.reshape(B, K)], axis=-1)
    return jnp.concatenate([top, jnp.asarray(rand_pts)], axis=1)
